# baseline (speedup 1.0000x reference)
import functools

import jax
import jax.numpy as jnp
from jax import lax
from jax.experimental import pallas as pl
from jax.experimental.pallas import tpu as pltpu
from jax.experimental.pallas import tpu_sc as plsc

CHUNK = 128
NBUF = 4


@functools.cache
def _make_gather(num_chunks: int, d: int):
    info = plsc.get_sparse_core_info()
    nw = info.num_cores * info.num_subcores
    assert num_chunks % (nw * NBUF) == 0
    cpw = num_chunks // nw

    mesh = plsc.VectorSubcoreMesh(core_axis_name="c", subcore_axis_name="s")

    @functools.partial(
        pl.kernel,
        mesh=mesh,
        out_type=jax.ShapeDtypeStruct((num_chunks * CHUNK, d), jnp.float32),
        scratch_types=[
            pltpu.VMEM((cpw, CHUNK), jnp.int32),
            pltpu.VMEM((NBUF, CHUNK, d), jnp.float32),
        ]
        + [pltpu.SemaphoreType.DMA] * NBUF,
        compiler_params=pltpu.CompilerParams(use_tc_tiling_on_sc=False),
    )
    def gather_kernel(table_hbm, idx_hbm, out_hbm, idx_v, rows_v, *sems):
        wid = lax.axis_index("s") * info.num_cores + lax.axis_index("c")
        chunk0 = wid * cpw

        pltpu.sync_copy(idx_hbm.at[pl.ds(chunk0, cpw)], idx_v)

        def start(c, b):
            pltpu.async_copy(table_hbm.at[idx_v.at[c]], rows_v.at[b], sems[b])

        for b in range(NBUF):
            start(b, b)

        def group(g, carry):
            for b in range(NBUF):
                c = g * NBUF + b
                pltpu.make_async_copy(
                    table_hbm.at[idx_v.at[c]], rows_v.at[b], sems[b]
                ).wait()
                pltpu.sync_copy(
                    rows_v.at[b], out_hbm.at[pl.ds((chunk0 + c) * CHUNK, CHUNK)]
                )

                @pl.when(c + NBUF < cpw)
                def _():
                    start(c + NBUF, b)

            return carry

        lax.fori_loop(0, cpw // NBUF, group, 0)

    return gather_kernel


def kernel(src, src_attn_mask, embedding_table):
    b, s = src.shape
    v, d = embedding_table.shape
    idx = src.reshape(-1, CHUNK).astype(jnp.int32)
    out = _make_gather(idx.shape[0], d)(embedding_table, idx)
    return out.reshape(b, s, d)
